# Initial kernel scaffold; baseline (speedup 1.0000x reference)
import jax
import jax.numpy as jnp
from jax.experimental import pallas as pl


def kernel(x, edge_index, W1, b1, g1, be1, a1, W2, b2, g2, be2, a2):
    raise NotImplementedError("write your pallas kernel here")


# speedup vs baseline: 7.9651x; 7.9651x over previous
import functools

import jax
import jax.numpy as jnp
from jax import lax
from jax.experimental import pallas as pl
from jax.experimental.pallas import tpu as pltpu
from jax.experimental.pallas import tpu_sc as plsc

_NC = 2
_NS = 16
_CH = 80
_K = 5


def _mesh():
    return plsc.VectorSubcoreMesh(core_axis_name="c", subcore_axis_name="s")


@functools.cache
def _make_deg(n, e):
    nch = e // (_NS * _CH)
    rpt = n // _NS
    assert e % (_NS * _CH) == 0 and n % _NS == 0 and nch % _K == 0

    @functools.partial(
        pl.kernel,
        mesh=_mesh(),
        out_type=jax.ShapeDtypeStruct((2, _NS, rpt, 16), jnp.float32),
        compiler_params=pltpu.CompilerParams(use_tc_tiling_on_sc=False),
        scratch_types=[
            pltpu.VMEM((nch, _CH), jnp.int32),
            pltpu.VMEM((_CH, 16), jnp.float32),
            pltpu.VMEM_SHARED((n, 16), jnp.float32),
        ] + [pltpu.SemaphoreType.DMA] * _K,
    )
    def deg_k(ei_hbm, zeros_hbm, ones_hbm, out_hbm, idx_v, ones_v, acc_sh, *sems):
        c = lax.axis_index("c")
        s = lax.axis_index("s")
        pltpu.sync_copy(ei_hbm.at[c, s], idx_v)
        pltpu.sync_copy(ones_hbm, ones_v)
        pltpu.sync_copy(zeros_hbm.at[s], acc_sh.at[pl.ds(s * rpt, rpt), :])
        plsc.subcore_barrier()

        def body(i, carry):
            descs = []
            for b in range(_K):
                descs.append(
                    pltpu.async_copy(
                        ones_v, acc_sh.at[idx_v.at[i * _K + b]], sems[b], add=True
                    )
                )
            for b in range(_K):
                descs[b].wait()
            return carry

        lax.fori_loop(0, nch // _K, body, 0)
        plsc.subcore_barrier()
        pltpu.sync_copy(acc_sh.at[pl.ds(s * rpt, rpt), :], out_hbm.at[c, s])

    return deg_k


@functools.cache
def _make_agg(n, e, d):
    hd = d // _NC
    nch = e // (_NS * _CH)
    rpt = n // _NS
    assert e % (_NS * _CH) == 0 and d % _NC == 0 and nch % _K == 0

    @functools.partial(
        pl.kernel,
        mesh=_mesh(),
        out_type=jax.ShapeDtypeStruct((2, _NS, rpt, hd), jnp.float32),
        compiler_params=pltpu.CompilerParams(use_tc_tiling_on_sc=False),
        scratch_types=[
            pltpu.VMEM((nch, _CH), jnp.int32),
            pltpu.VMEM((nch, _CH), jnp.int32),
            pltpu.VMEM((_K, _CH, hd), jnp.float32),
            pltpu.VMEM_SHARED((n, hd), jnp.float32),
        ] + [pltpu.SemaphoreType.DMA] * _K,
    )
    def agg_k(h_hbm, src_hbm, dst_hbm, zeros_hbm, out_hbm,
              sidx, didx, rows, acc_sh, *sems):
        c = lax.axis_index("c")
        s = lax.axis_index("s")
        pltpu.sync_copy(src_hbm.at[c, s], sidx)
        pltpu.sync_copy(dst_hbm.at[s], didx)
        pltpu.sync_copy(zeros_hbm.at[s], acc_sh.at[pl.ds(s * rpt, rpt), :])
        plsc.subcore_barrier()

        def body(i, carry):
            descs = []
            for b in range(_K):
                descs.append(
                    pltpu.async_copy(
                        h_hbm.at[sidx.at[i * _K + b]], rows.at[b], sems[b]
                    )
                )
            for b in range(_K):
                descs[b].wait()
                pltpu.sync_copy(rows.at[b], acc_sh.at[didx.at[i * _K + b]], add=True)
            return carry

        lax.fori_loop(0, nch // _K, body, 0)
        plsc.subcore_barrier()
        pltpu.sync_copy(acc_sh.at[pl.ds(s * rpt, rpt), :], out_hbm.at[c, s])

    return agg_k


def _prep_call(x, dsrc):
    n, d = x.shape
    hd = d // 2

    def body(x_ref, d_ref, o_ref):
        nsrc = lax.rsqrt(jnp.maximum(d_ref[:, 0:1], 1.0))
        h = x_ref[:, :] * nsrc
        o_ref[0] = h[:, :hd]
        o_ref[1] = h[:, hd:]

    return pl.pallas_call(
        body, out_shape=jax.ShapeDtypeStruct((2, n, hd), jnp.float32)
    )(x, dsrc)


def _dense_call(p0, p1, ddst, dsrc, w, b, g, be, al, split_out):
    n, hd = p0.shape
    d = 2 * hd

    def body(p0r, p1r, ddr, dsr, wr, br, gr, ber, alr, o_ref):
        nd = lax.rsqrt(jnp.maximum(ddr[:, 0:1], 1.0))
        agg = jnp.concatenate([p0r[:, :], p1r[:, :]], axis=1) * nd
        y = jnp.dot(agg, wr[:, :], preferred_element_type=jnp.float32) + br[:, :]
        m = jnp.mean(y, axis=0, keepdims=True)
        yc = y - m
        v = jnp.mean(yc * yc, axis=0, keepdims=True)
        y = gr[:, :] * yc * lax.rsqrt(v + 1e-5) + ber[:, :]
        y = jnp.where(y >= 0.0, y, alr[0, 0] * y)
        if split_out:
            y = y * lax.rsqrt(jnp.maximum(dsr[:, 0:1], 1.0))
            o_ref[0] = y[:, :hd]
            o_ref[1] = y[:, hd:]
        else:
            o_ref[:, :] = y

    out_shape = (2, n, hd) if split_out else (n, d)
    return pl.pallas_call(
        body, out_shape=jax.ShapeDtypeStruct(out_shape, jnp.float32)
    )(p0, p1, ddst, dsrc, w, b, g, be, al)


def kernel(x, edge_index, W1, b1, g1, be1, a1, W2, b2, g2, be2, a2):
    n, d = x.shape
    e = edge_index.shape[1]
    nch = e // (_NS * _CH)
    rpt = n // _NS

    src3 = edge_index[0].reshape(_NS, nch, _CH)
    dst3 = edge_index[1].reshape(_NS, nch, _CH)
    ei4 = jnp.stack([src3, dst3])
    src4 = jnp.stack([src3, src3 + n])
    zeros_h = jnp.zeros((_NS, rpt, d // _NC), jnp.float32)
    zeros16 = jnp.zeros((_NS, rpt, 16), jnp.float32)
    ones16 = jnp.ones((_CH, 16), jnp.float32)

    deg = _make_deg(n, e)(ei4, zeros16, ones16).reshape(2, n, 16)
    dsrc = deg[0]
    ddst = deg[1]

    b1r, g1r, be1r = b1.reshape(1, d), g1.reshape(1, d), be1.reshape(1, d)
    b2r, g2r, be2r = b2.reshape(1, d), g2.reshape(1, d), be2.reshape(1, d)
    a1r, a2r = a1.reshape(1, 1), a2.reshape(1, 1)

    agg = _make_agg(n, e, d)
    h = _prep_call(x, dsrc).reshape(2 * n, d // _NC)
    p = agg(h, src4, dst3, zeros_h).reshape(2, n, d // _NC)
    h = _dense_call(p[0], p[1], ddst, dsrc, W1, b1r, g1r, be1r, a1r, True)
    h = h.reshape(2 * n, d // _NC)
    p = agg(h, src4, dst3, zeros_h).reshape(2, n, d // _NC)
    out = _dense_call(p[0], p[1], ddst, dsrc, W2, b2r, g2r, be2r, a2r, False)
    return out


# speedup vs baseline: 9.8401x; 1.2354x over previous
import functools

import jax
import jax.numpy as jnp
from jax import lax
from jax.experimental import pallas as pl
from jax.experimental.pallas import tpu as pltpu
from jax.experimental.pallas import tpu_sc as plsc

_NC = 2
_NS = 16
_CH = 125
_K = 5


def _mesh():
    return plsc.VectorSubcoreMesh(core_axis_name="c", subcore_axis_name="s")


@functools.cache
def _make_deg(n, e):
    nch = e // (_NS * _CH)
    rpt = n // _NS
    assert e % (_NS * _CH) == 0 and n % _NS == 0 and nch % _K == 0
    @functools.partial(
        pl.kernel,
        mesh=_mesh(),
        out_type=jax.ShapeDtypeStruct((2, _NS, rpt, 16), jnp.float32),
        compiler_params=pltpu.CompilerParams(use_tc_tiling_on_sc=False),
        scratch_types=[
            pltpu.VMEM((nch, _CH), jnp.int32),
            pltpu.VMEM((_CH, 16), jnp.float32),
            pltpu.VMEM_SHARED((n, 16), jnp.float32),
        ] + [pltpu.SemaphoreType.DMA] * _K,
    )
    def deg_k(ei_hbm, zeros_hbm, ones_hbm, out_hbm, idx_v, ones_v,
              acc_sh, *sems):
        c = lax.axis_index("c")
        s = lax.axis_index("s")
        pltpu.sync_copy(ei_hbm.at[c, s], idx_v)
        pltpu.sync_copy(ones_hbm, ones_v)
        pltpu.sync_copy(zeros_hbm.at[s], acc_sh.at[pl.ds(s * rpt, rpt), :])
        plsc.subcore_barrier()

        def body(i, carry):
            descs = []
            for b in range(_K):
                descs.append(
                    pltpu.async_copy(
                        ones_v, acc_sh.at[idx_v.at[i * _K + b]], sems[b], add=True
                    )
                )
            for b in range(_K):
                descs[b].wait()
            return carry

        lax.fori_loop(0, nch // _K, body, 0)
        plsc.subcore_barrier()
        pltpu.sync_copy(acc_sh.at[pl.ds(s * rpt, rpt), :], out_hbm.at[c, s])

    return deg_k


@functools.cache
def _make_agg(n, e, d):
    hd = d // _NC
    nch = e // (_NS * _CH)
    rpt = n // _NS
    niter = nch // _K
    assert e % (_NS * _CH) == 0 and d % _NC == 0 and nch % _K == 0

    @functools.partial(
        pl.kernel,
        mesh=_mesh(),
        out_type=jax.ShapeDtypeStruct((2, _NS, rpt, hd), jnp.float32),
        compiler_params=pltpu.CompilerParams(use_tc_tiling_on_sc=False),
        scratch_types=[
            pltpu.VMEM((nch, _CH), jnp.int32),
            pltpu.VMEM((nch, _CH), jnp.int32),
            pltpu.VMEM((_K, _CH, hd), jnp.float32),
            pltpu.VMEM_SHARED((n, hd), jnp.float32),
        ] + [pltpu.SemaphoreType.DMA] * (2 * _K),
    )
    def agg_k(h_hbm, src_hbm, dst_hbm, zeros_hbm, out_hbm,
              sidx, didx, rows, acc_sh, *sems):
        gsem = sems[:_K]
        ssem = sems[_K:]
        c = lax.axis_index("c")
        s = lax.axis_index("s")
        pltpu.sync_copy(src_hbm.at[c, s], sidx)
        pltpu.sync_copy(dst_hbm.at[s], didx)
        pltpu.sync_copy(zeros_hbm.at[s], acc_sh.at[pl.ds(s * rpt, rpt), :])
        plsc.subcore_barrier()

        for b in range(_K):
            pltpu.async_copy(h_hbm.at[sidx.at[b]], rows.at[b], gsem[b])

        def body(i, carry):
            for b in range(_K):
                k = i * _K + b
                pltpu.make_async_copy(
                    h_hbm.at[sidx.at[k]], rows.at[b], gsem[b]
                ).wait()
                pltpu.async_copy(
                    rows.at[b], acc_sh.at[didx.at[k]], ssem[b], add=True
                )
            for b in range(_K):
                k = i * _K + b
                pltpu.make_async_copy(
                    rows.at[b], acc_sh.at[didx.at[k]], ssem[b]
                ).wait()
                @pl.when(i + 1 < niter)
                def _():
                    pltpu.async_copy(
                        h_hbm.at[sidx.at[k + _K]], rows.at[b], gsem[b]
                    )
            return carry

        lax.fori_loop(0, niter, body, 0)
        plsc.subcore_barrier()
        pltpu.sync_copy(acc_sh.at[pl.ds(s * rpt, rpt), :], out_hbm.at[c, s])

    return agg_k


def _prep_call(x, dsrc):
    n, d = x.shape
    hd = d // 2

    def body(x_ref, d_ref, o_ref):
        nsrc = lax.rsqrt(jnp.maximum(d_ref[:, 0:1], 1.0))
        h = x_ref[:, :] * nsrc
        o_ref[0] = h[:, :hd]
        o_ref[1] = h[:, hd:]

    return pl.pallas_call(
        body, out_shape=jax.ShapeDtypeStruct((2, n, hd), jnp.float32)
    )(x, dsrc)


def _dense_call(p0, p1, ddst, dsrc, w, b, g, be, al, split_out):
    n, hd = p0.shape
    d = 2 * hd

    def body(p0r, p1r, ddr, dsr, wr, br, gr, ber, alr, o_ref):
        nd = lax.rsqrt(jnp.maximum(ddr[:, 0:1], 1.0))
        agg = jnp.concatenate([p0r[:, :], p1r[:, :]], axis=1) * nd
        y = jnp.dot(agg, wr[:, :], preferred_element_type=jnp.float32) + br[:, :]
        m = jnp.mean(y, axis=0, keepdims=True)
        yc = y - m
        v = jnp.mean(yc * yc, axis=0, keepdims=True)
        y = gr[:, :] * yc * lax.rsqrt(v + 1e-5) + ber[:, :]
        y = jnp.where(y >= 0.0, y, alr[0, 0] * y)
        if split_out:
            y = y * lax.rsqrt(jnp.maximum(dsr[:, 0:1], 1.0))
            o_ref[0] = y[:, :hd]
            o_ref[1] = y[:, hd:]
        else:
            o_ref[:, :] = y

    out_shape = (2, n, hd) if split_out else (n, d)
    return pl.pallas_call(
        body, out_shape=jax.ShapeDtypeStruct(out_shape, jnp.float32)
    )(p0, p1, ddst, dsrc, w, b, g, be, al)


def kernel(x, edge_index, W1, b1, g1, be1, a1, W2, b2, g2, be2, a2):
    n, d = x.shape
    e = edge_index.shape[1]
    nch = e // (_NS * _CH)
    rpt = n // _NS
    hd = d // _NC

    src3 = edge_index[0].reshape(_NS, nch, _CH)
    dst3 = edge_index[1].reshape(_NS, nch, _CH)
    ei4 = jnp.stack([src3, dst3])
    src4 = jnp.stack([src3, src3 + n])
    zeros_h = jnp.zeros((_NS, rpt, hd), jnp.float32)
    zeros16 = jnp.zeros((_NS, rpt, 16), jnp.float32)
    ones16 = jnp.ones((_CH, 16), jnp.float32)

    deg = _make_deg(n, e)(ei4, zeros16, ones16).reshape(2, n, 16)
    dsrc = deg[0]
    ddst = deg[1]

    b1r, g1r, be1r = b1.reshape(1, d), g1.reshape(1, d), be1.reshape(1, d)
    b2r, g2r, be2r = b2.reshape(1, d), g2.reshape(1, d), be2.reshape(1, d)
    a1r, a2r = a1.reshape(1, 1), a2.reshape(1, 1)

    agg = _make_agg(n, e, d)
    h = _prep_call(x, dsrc).reshape(2 * n, hd)
    p = agg(h, src4, dst3, zeros_h).reshape(2, n, hd)
    h = _dense_call(p[0], p[1], ddst, dsrc,
                    W1, b1r, g1r, be1r, a1r, True)
    h = h.reshape(2 * n, hd)
    p = agg(h, src4, dst3, zeros_h).reshape(2, n, hd)
    out = _dense_call(p[0], p[1], ddst, dsrc,
                      W2, b2r, g2r, be2r, a2r, False)
    return out


# speedup vs baseline: 9.9791x; 1.0141x over previous
import functools

import jax
import jax.numpy as jnp
from jax import lax
from jax.experimental import pallas as pl
from jax.experimental.pallas import tpu as pltpu
from jax.experimental.pallas import tpu_sc as plsc

_NC = 2
_NS = 16
_CH = 125
_K = 5


def _mesh():
    return plsc.VectorSubcoreMesh(core_axis_name="c", subcore_axis_name="s")


@functools.cache
def _make_deg(n, e):
    nch = e // (_NS * _CH)
    rpt = n // _NS
    assert e % (_NS * _CH) == 0 and n % _NS == 0 and nch % _K == 0
    @functools.partial(
        pl.kernel,
        mesh=_mesh(),
        out_type=jax.ShapeDtypeStruct((2, _NS, rpt, 16), jnp.float32),
        compiler_params=pltpu.CompilerParams(use_tc_tiling_on_sc=False),
        scratch_types=[
            pltpu.VMEM((nch, _CH), jnp.int32),
            pltpu.VMEM((_CH, 16), jnp.float32),
            pltpu.VMEM_SHARED((n, 16), jnp.float32),
        ] + [pltpu.SemaphoreType.DMA] * _K,
    )
    def deg_k(ei_hbm, zeros_hbm, ones_hbm, out_hbm, idx_v, ones_v,
              acc_sh, *sems):
        c = lax.axis_index("c")
        s = lax.axis_index("s")
        pltpu.sync_copy(ei_hbm.at[c, s], idx_v)
        pltpu.sync_copy(ones_hbm, ones_v)
        pltpu.sync_copy(zeros_hbm.at[s], acc_sh.at[pl.ds(s * rpt, rpt), :])
        plsc.subcore_barrier()

        def body(i, carry):
            descs = []
            for b in range(_K):
                descs.append(
                    pltpu.async_copy(
                        ones_v, acc_sh.at[idx_v.at[i * _K + b]], sems[b], add=True
                    )
                )
            for b in range(_K):
                descs[b].wait()
            return carry

        lax.fori_loop(0, nch // _K, body, 0)
        plsc.subcore_barrier()
        pltpu.sync_copy(acc_sh.at[pl.ds(s * rpt, rpt), :], out_hbm.at[c, s])

    return deg_k


@functools.cache
def _make_agg(n, e, d):
    hd = d // _NC
    nch = e // (_NS * _CH)
    rpt = n // _NS
    niter = nch // _K
    assert e % (_NS * _CH) == 0 and d % _NC == 0 and nch % _K == 0

    @functools.partial(
        pl.kernel,
        mesh=_mesh(),
        out_type=jax.ShapeDtypeStruct((2, _NS, rpt, hd), jnp.float32),
        compiler_params=pltpu.CompilerParams(use_tc_tiling_on_sc=False),
        scratch_types=[
            pltpu.VMEM((nch, _CH), jnp.int32),
            pltpu.VMEM((nch, _CH), jnp.int32),
            pltpu.VMEM((_K, _CH, hd), jnp.float32),
            pltpu.VMEM_SHARED((n, hd), jnp.float32),
        ] + [pltpu.SemaphoreType.DMA] * (2 * _K),
    )
    def agg_k(h_hbm, src_hbm, dst_hbm, zeros_hbm, out_hbm,
              sidx, didx, rows, acc_sh, *sems):
        gsem = sems[:_K]
        ssem = sems[_K:]
        c = lax.axis_index("c")
        s = lax.axis_index("s")
        h_view = h_hbm.at[pl.ds(c * n, n)]
        pltpu.sync_copy(src_hbm.at[s], sidx)
        pltpu.sync_copy(dst_hbm.at[s], didx)
        pltpu.sync_copy(zeros_hbm.at[s], acc_sh.at[pl.ds(s * rpt, rpt), :])
        plsc.subcore_barrier()

        for b in range(_K):
            pltpu.async_copy(h_view.at[sidx.at[b]], rows.at[b], gsem[b])

        def body(i, carry):
            for b in range(_K):
                k = i * _K + b
                pltpu.make_async_copy(
                    h_view.at[sidx.at[k]], rows.at[b], gsem[b]
                ).wait()
                pltpu.async_copy(
                    rows.at[b], acc_sh.at[didx.at[k]], ssem[b], add=True
                )
            for b in range(_K):
                k = i * _K + b
                pltpu.make_async_copy(
                    rows.at[b], acc_sh.at[didx.at[k]], ssem[b]
                ).wait()
                @pl.when(i + 1 < niter)
                def _():
                    pltpu.async_copy(
                        h_view.at[sidx.at[k + _K]], rows.at[b], gsem[b]
                    )
            return carry

        lax.fori_loop(0, niter, body, 0)
        plsc.subcore_barrier()
        pltpu.sync_copy(acc_sh.at[pl.ds(s * rpt, rpt), :], out_hbm.at[c, s])

    return agg_k


def _prep_call(x, dsrc):
    n, d = x.shape
    hd = d // 2

    def body(x_ref, d_ref, o_ref):
        nsrc = lax.rsqrt(jnp.maximum(d_ref[:, 0:1], 1.0))
        h = x_ref[:, :] * nsrc
        o_ref[0] = h[:, :hd]
        o_ref[1] = h[:, hd:]

    return pl.pallas_call(
        body, out_shape=jax.ShapeDtypeStruct((2, n, hd), jnp.float32)
    )(x, dsrc)


def _dense_call(p0, p1, ddst, dsrc, w, b, g, be, al, split_out):
    n, hd = p0.shape
    d = 2 * hd

    def body(p0r, p1r, ddr, dsr, wr, br, gr, ber, alr, o_ref):
        nd = lax.rsqrt(jnp.maximum(ddr[:, 0:1], 1.0))
        agg = jnp.concatenate([p0r[:, :], p1r[:, :]], axis=1) * nd
        y = jnp.dot(agg, wr[:, :], preferred_element_type=jnp.float32) + br[:, :]
        m = jnp.mean(y, axis=0, keepdims=True)
        yc = y - m
        v = jnp.mean(yc * yc, axis=0, keepdims=True)
        y = gr[:, :] * yc * lax.rsqrt(v + 1e-5) + ber[:, :]
        y = jnp.where(y >= 0.0, y, alr[0, 0] * y)
        if split_out:
            y = y * lax.rsqrt(jnp.maximum(dsr[:, 0:1], 1.0))
            o_ref[0] = y[:, :hd]
            o_ref[1] = y[:, hd:]
        else:
            o_ref[:, :] = y

    out_shape = (2, n, hd) if split_out else (n, d)
    return pl.pallas_call(
        body, out_shape=jax.ShapeDtypeStruct(out_shape, jnp.float32)
    )(p0, p1, ddst, dsrc, w, b, g, be, al)


def kernel(x, edge_index, W1, b1, g1, be1, a1, W2, b2, g2, be2, a2):
    n, d = x.shape
    e = edge_index.shape[1]
    nch = e // (_NS * _CH)
    rpt = n // _NS
    hd = d // _NC

    src3 = edge_index[0].reshape(_NS, nch, _CH)
    dst3 = edge_index[1].reshape(_NS, nch, _CH)
    ei4 = jnp.stack([src3, dst3])
    zeros_h = jnp.zeros((_NS, rpt, hd), jnp.float32)
    zeros16 = jnp.zeros((_NS, rpt, 16), jnp.float32)
    ones16 = jnp.ones((_CH, 16), jnp.float32)

    deg = _make_deg(n, e)(ei4, zeros16, ones16).reshape(2, n, 16)
    dsrc = deg[0]
    ddst = deg[1]

    b1r, g1r, be1r = b1.reshape(1, d), g1.reshape(1, d), be1.reshape(1, d)
    b2r, g2r, be2r = b2.reshape(1, d), g2.reshape(1, d), be2.reshape(1, d)
    a1r, a2r = a1.reshape(1, 1), a2.reshape(1, 1)

    agg = _make_agg(n, e, d)
    h = _prep_call(x, dsrc).reshape(2 * n, hd)
    p = agg(h, src3, dst3, zeros_h).reshape(2, n, hd)
    h = _dense_call(p[0], p[1], ddst, dsrc,
                    W1, b1r, g1r, be1r, a1r, True)
    h = h.reshape(2 * n, hd)
    p = agg(h, src3, dst3, zeros_h).reshape(2, n, hd)
    out = _dense_call(p[0], p[1], ddst, dsrc,
                      W2, b2r, g2r, be2r, a2r, False)
    return out


# speedup vs baseline: 11.5997x; 1.1624x over previous
import functools

import jax
import jax.numpy as jnp
from jax import lax
from jax.experimental import pallas as pl
from jax.experimental.pallas import tpu as pltpu
from jax.experimental.pallas import tpu_sc as plsc

_NC = 2
_NS = 16
_CH = 125
_K = 5


def _mesh():
    return plsc.VectorSubcoreMesh(core_axis_name="c", subcore_axis_name="s")


@functools.cache
def _make_deg(n, e):
    nch = e // (_NS * _CH)
    rpt = n // _NS
    assert e % (_NS * _CH) == 0 and n % _NS == 0 and nch % _K == 0
    @functools.partial(
        pl.kernel,
        mesh=_mesh(),
        out_type=jax.ShapeDtypeStruct((2, n, 16), jnp.float32),
        compiler_params=pltpu.CompilerParams(use_tc_tiling_on_sc=False),
        scratch_types=[
            pltpu.VMEM((nch, _CH), jnp.int32),
            pltpu.VMEM((_CH, 16), jnp.float32),
            pltpu.VMEM_SHARED((n, 16), jnp.float32),
        ] + [pltpu.SemaphoreType.DMA] * _K,
    )
    def deg_k(ei_hbm, zeros_hbm, ones_hbm, out_hbm, idx_v, ones_v,
              acc_sh, *sems):
        c = lax.axis_index("c")
        s = lax.axis_index("s")
        pltpu.sync_copy(ei_hbm.at[c, s], idx_v)
        pltpu.sync_copy(ones_hbm, ones_v)
        pltpu.sync_copy(zeros_hbm.at[s], acc_sh.at[pl.ds(s * rpt, rpt), :])
        plsc.subcore_barrier()

        def body(i, carry):
            descs = []
            for b in range(_K):
                descs.append(
                    pltpu.async_copy(
                        ones_v, acc_sh.at[idx_v.at[i * _K + b]], sems[b], add=True
                    )
                )
            for b in range(_K):
                descs[b].wait()
            return carry

        lax.fori_loop(0, nch // _K, body, 0)
        plsc.subcore_barrier()
        pltpu.sync_copy(acc_sh.at[pl.ds(s * rpt, rpt), :],
                        out_hbm.at[c, pl.ds(s * rpt, rpt), :])

    return deg_k


@functools.cache
def _make_agg(n, e, d):
    hd = d // _NC
    nch = e // (_NS * _CH)
    rpt = n // _NS
    niter = nch // _K
    assert e % (_NS * _CH) == 0 and d % _NC == 0 and nch % _K == 0

    @functools.partial(
        pl.kernel,
        mesh=_mesh(),
        out_type=jax.ShapeDtypeStruct((2, n, hd), jnp.float32),
        compiler_params=pltpu.CompilerParams(use_tc_tiling_on_sc=False),
        scratch_types=[
            pltpu.VMEM((nch, _CH), jnp.int32),
            pltpu.VMEM((nch, _CH), jnp.int32),
            pltpu.VMEM((_K, _CH, hd), jnp.float32),
            pltpu.VMEM_SHARED((n, hd), jnp.float32),
        ] + [pltpu.SemaphoreType.DMA] * (2 * _K),
    )
    def agg_k(h_hbm, ei_hbm, zeros_hbm, out_hbm,
              sidx, didx, rows, acc_sh, *sems):
        gsem = sems[:_K]
        ssem = sems[_K:]
        c = lax.axis_index("c")
        s = lax.axis_index("s")
        h_view = h_hbm.at[c]
        pltpu.sync_copy(ei_hbm.at[0, s], sidx)
        pltpu.sync_copy(ei_hbm.at[1, s], didx)
        pltpu.sync_copy(zeros_hbm.at[s], acc_sh.at[pl.ds(s * rpt, rpt), :])
        plsc.subcore_barrier()

        for b in range(_K):
            pltpu.async_copy(h_view.at[sidx.at[b]], rows.at[b], gsem[b])

        def body(i, carry):
            for b in range(_K):
                k = i * _K + b
                pltpu.make_async_copy(
                    h_view.at[sidx.at[k]], rows.at[b], gsem[b]
                ).wait()
                pltpu.async_copy(
                    rows.at[b], acc_sh.at[didx.at[k]], ssem[b], add=True
                )
            for b in range(_K):
                k = i * _K + b
                pltpu.make_async_copy(
                    rows.at[b], acc_sh.at[didx.at[k]], ssem[b]
                ).wait()
                @pl.when(i + 1 < niter)
                def _():
                    pltpu.async_copy(
                        h_view.at[sidx.at[k + _K]], rows.at[b], gsem[b]
                    )
            return carry

        lax.fori_loop(0, niter, body, 0)
        plsc.subcore_barrier()
        pltpu.sync_copy(acc_sh.at[pl.ds(s * rpt, rpt), :],
                        out_hbm.at[c, pl.ds(s * rpt, rpt), :])

    return agg_k


def _prep_call(x, deg):
    n, d = x.shape
    hd = d // 2

    def body(x_ref, d_ref, o_ref):
        nsrc = lax.rsqrt(jnp.maximum(d_ref[0][:, 0:1], 1.0))
        h = x_ref[:, :] * nsrc
        o_ref[0] = h[:, :hd]
        o_ref[1] = h[:, hd:]

    return pl.pallas_call(
        body, out_shape=jax.ShapeDtypeStruct((2, n, hd), jnp.float32)
    )(x, deg)


def _dense_call(p, deg, w, b, g, be, al, split_out):
    _, n, hd = p.shape
    d = 2 * hd

    def body(p_ref, d_ref, wr, br, gr, ber, alr, o_ref):
        nd = lax.rsqrt(jnp.maximum(d_ref[1][:, 0:1], 1.0))
        y = (
            jnp.dot(p_ref[0] * nd, wr[:hd, :],
                    preferred_element_type=jnp.float32)
            + jnp.dot(p_ref[1] * nd, wr[hd:, :],
                      preferred_element_type=jnp.float32)
            + br[:, :]
        )
        m = jnp.mean(y, axis=0, keepdims=True)
        yc = y - m
        v = jnp.mean(yc * yc, axis=0, keepdims=True)
        y = gr[:, :] * yc * lax.rsqrt(v + 1e-5) + ber[:, :]
        y = jnp.where(y >= 0.0, y, alr[0, 0] * y)
        if split_out:
            y = y * lax.rsqrt(jnp.maximum(d_ref[0][:, 0:1], 1.0))
            o_ref[0] = y[:, :hd]
            o_ref[1] = y[:, hd:]
        else:
            o_ref[:, :] = y

    out_shape = (2, n, hd) if split_out else (n, d)
    return pl.pallas_call(
        body, out_shape=jax.ShapeDtypeStruct(out_shape, jnp.float32)
    )(p, deg, w, b, g, be, al)


def kernel(x, edge_index, W1, b1, g1, be1, a1, W2, b2, g2, be2, a2):
    n, d = x.shape
    e = edge_index.shape[1]
    nch = e // (_NS * _CH)
    rpt = n // _NS
    hd = d // _NC

    ei4 = jnp.reshape(edge_index, (2, _NS, nch, _CH))
    zeros_h = jnp.zeros((_NS, rpt, hd), jnp.float32)
    zeros16 = jnp.zeros((_NS, rpt, 16), jnp.float32)
    ones16 = jnp.ones((_CH, 16), jnp.float32)

    deg = _make_deg(n, e)(ei4, zeros16, ones16)

    b1r, g1r, be1r = b1.reshape(1, d), g1.reshape(1, d), be1.reshape(1, d)
    b2r, g2r, be2r = b2.reshape(1, d), g2.reshape(1, d), be2.reshape(1, d)
    a1r, a2r = a1.reshape(1, 1), a2.reshape(1, 1)

    agg = _make_agg(n, e, d)
    h = _prep_call(x, deg)
    p = agg(h, ei4, zeros_h)
    h = _dense_call(p, deg, W1, b1r, g1r, be1r, a1r, True)
    p = agg(h, ei4, zeros_h)
    out = _dense_call(p, deg, W2, b2r, g2r, be2r, a2r, False)
    return out


# speedup vs baseline: 12.4896x; 1.0767x over previous
import functools

import jax
import jax.numpy as jnp
from jax import lax
from jax.experimental import pallas as pl
from jax.experimental.pallas import tpu as pltpu
from jax.experimental.pallas import tpu_sc as plsc

_NC = 2
_NS = 16
_CH = 125
_K = 5


def _mesh():
    return plsc.VectorSubcoreMesh(core_axis_name="c", subcore_axis_name="s")


@functools.cache
def _make_deg(n, e):
    nch = e // (_NS * _CH)
    rpt = n // _NS
    assert e % (_NS * _CH) == 0 and n % _NS == 0 and nch % _K == 0
    @functools.partial(
        pl.kernel,
        mesh=_mesh(),
        out_type=jax.ShapeDtypeStruct((2, n, 16), jnp.float32),
        compiler_params=pltpu.CompilerParams(use_tc_tiling_on_sc=False),
        scratch_types=[
            pltpu.VMEM((nch, _CH), jnp.int32),
            pltpu.VMEM((_CH, 16), jnp.float32),
            pltpu.VMEM_SHARED((n, 16), jnp.float32),
        ] + [pltpu.SemaphoreType.DMA] * _K,
    )
    def deg_k(ei_hbm, zeros_hbm, ones_hbm, out_hbm, idx_v, ones_v,
              acc_sh, *sems):
        c = lax.axis_index("c")
        s = lax.axis_index("s")
        pltpu.sync_copy(ei_hbm.at[c, s], idx_v)
        pltpu.sync_copy(ones_hbm, ones_v)
        pltpu.sync_copy(zeros_hbm.at[s], acc_sh.at[pl.ds(s * rpt, rpt), :])
        plsc.subcore_barrier()

        def body(i, carry):
            descs = []
            for b in range(_K):
                descs.append(
                    pltpu.async_copy(
                        ones_v, acc_sh.at[idx_v.at[i * _K + b]], sems[b], add=True
                    )
                )
            for b in range(_K):
                descs[b].wait()
            return carry

        lax.fori_loop(0, nch // _K, body, 0)
        plsc.subcore_barrier()
        pltpu.sync_copy(acc_sh.at[pl.ds(s * rpt, rpt), :],
                        out_hbm.at[c, pl.ds(s * rpt, rpt), :])

    return deg_k


@functools.cache
def _make_agg(n, e, d):
    hd = d // _NC
    nch = e // (_NS * _CH)
    rpt = n // _NS
    niter = nch // _K
    assert e % (_NS * _CH) == 0 and d % _NC == 0 and nch % _K == 0

    @functools.partial(
        pl.kernel,
        mesh=_mesh(),
        out_type=jax.ShapeDtypeStruct((2, n, hd), jnp.float32),
        compiler_params=pltpu.CompilerParams(use_tc_tiling_on_sc=False),
        scratch_types=[
            pltpu.VMEM((nch, _CH), jnp.int32),
            pltpu.VMEM((nch, _CH), jnp.int32),
            pltpu.VMEM((_K, _CH, hd), jnp.float32),
            pltpu.VMEM_SHARED((n, hd), jnp.float32),
        ] + [pltpu.SemaphoreType.DMA] * (2 * _K),
    )
    def agg_k(h_hbm, si_hbm, ei_hbm, zeros_hbm, out_hbm,
              sidx, didx, rows, acc_sh, *sems):
        gsem = sems[:_K]
        ssem = sems[_K:]
        c = lax.axis_index("c")
        s = lax.axis_index("s")
        h_view = h_hbm
        pltpu.sync_copy(si_hbm.at[c, s], sidx)
        pltpu.sync_copy(ei_hbm.at[1, s], didx)
        pltpu.sync_copy(zeros_hbm.at[s], acc_sh.at[pl.ds(s * rpt, rpt), :])
        plsc.subcore_barrier()

        for b in range(_K):
            pltpu.async_copy(h_view.at[sidx.at[b]], rows.at[b], gsem[b])

        def body(i, carry):
            for b in range(_K):
                k = i * _K + b
                pltpu.make_async_copy(
                    h_view.at[sidx.at[k]], rows.at[b], gsem[b]
                ).wait()
                pltpu.async_copy(
                    rows.at[b], acc_sh.at[didx.at[k]], ssem[b], add=True
                )
            for b in range(_K):
                k = i * _K + b
                pltpu.make_async_copy(
                    rows.at[b], acc_sh.at[didx.at[k]], ssem[b]
                ).wait()
                @pl.when(i + 1 < niter)
                def _():
                    pltpu.async_copy(
                        h_view.at[sidx.at[k + _K]], rows.at[b], gsem[b]
                    )
            return carry

        lax.fori_loop(0, niter, body, 0)
        plsc.subcore_barrier()
        pltpu.sync_copy(acc_sh.at[pl.ds(s * rpt, rpt), :],
                        out_hbm.at[c, pl.ds(s * rpt, rpt), :])

    return agg_k


def _prep_call(x, deg):
    n, d = x.shape

    def body(x_ref, d_ref, o_ref):
        nsrc = lax.rsqrt(jnp.maximum(d_ref[0][:, 0:1], 1.0))
        o_ref[:, :] = x_ref[:, :] * nsrc

    return pl.pallas_call(
        body, out_shape=jax.ShapeDtypeStruct((n, d), jnp.float32)
    )(x, deg)


def _dense_call(p, deg, w, b, g, be, al, split_out):
    _, n, hd = p.shape
    d = 2 * hd

    def body(p_ref, d_ref, wr, br, gr, ber, alr, o_ref):
        nd = lax.rsqrt(jnp.maximum(d_ref[1][:, 0:1], 1.0))
        y = (
            jnp.dot(p_ref[0] * nd, wr[:hd, :],
                    preferred_element_type=jnp.float32)
            + jnp.dot(p_ref[1] * nd, wr[hd:, :],
                      preferred_element_type=jnp.float32)
            + br[:, :]
        )
        m = jnp.mean(y, axis=0, keepdims=True)
        yc = y - m
        v = jnp.mean(yc * yc, axis=0, keepdims=True)
        y = gr[:, :] * yc * lax.rsqrt(v + 1e-5) + ber[:, :]
        y = jnp.where(y >= 0.0, y, alr[0, 0] * y)
        if split_out:
            y = y * lax.rsqrt(jnp.maximum(d_ref[0][:, 0:1], 1.0))
        o_ref[:, :] = y

    return pl.pallas_call(
        body, out_shape=jax.ShapeDtypeStruct((n, d), jnp.float32)
    )(p, deg, w, b, g, be, al)


def kernel(x, edge_index, W1, b1, g1, be1, a1, W2, b2, g2, be2, a2):
    n, d = x.shape
    e = edge_index.shape[1]
    nch = e // (_NS * _CH)
    rpt = n // _NS
    hd = d // _NC

    ei4 = jnp.reshape(edge_index, (2, _NS, nch, _CH))
    s2 = ei4[0] * 2
    si4 = jnp.stack([s2, s2 + 1])
    zeros_h = jnp.zeros((_NS, rpt, hd), jnp.float32)
    zeros16 = jnp.zeros((_NS, rpt, 16), jnp.float32)
    ones16 = jnp.ones((_CH, 16), jnp.float32)

    deg = _make_deg(n, e)(ei4, zeros16, ones16)

    b1r, g1r, be1r = b1.reshape(1, d), g1.reshape(1, d), be1.reshape(1, d)
    b2r, g2r, be2r = b2.reshape(1, d), g2.reshape(1, d), be2.reshape(1, d)
    a1r, a2r = a1.reshape(1, 1), a2.reshape(1, 1)

    agg = _make_agg(n, e, d)
    h = _prep_call(x, deg).reshape(_NC * n, hd)
    p = agg(h, si4, ei4, zeros_h)
    h = _dense_call(p, deg, W1, b1r, g1r, be1r, a1r, True).reshape(_NC * n, hd)
    p = agg(h, si4, ei4, zeros_h)
    out = _dense_call(p, deg, W2, b2r, g2r, be2r, a2r, False)
    return out


# speedup vs baseline: 13.3712x; 1.0706x over previous
import functools

import jax
import jax.numpy as jnp
from jax import lax
from jax.experimental import pallas as pl
from jax.experimental.pallas import tpu as pltpu
from jax.experimental.pallas import tpu_sc as plsc

_NC = 2
_NS = 16
_CH = 125
_K = 5


def _mesh():
    return plsc.VectorSubcoreMesh(core_axis_name="c", subcore_axis_name="s")


@functools.cache
def _make_deg(n, e):
    nch = e // (_NS * _CH)
    rpt = n // _NS
    assert e % (_NS * _CH) == 0 and n % _NS == 0 and nch % _K == 0
    @functools.partial(
        pl.kernel,
        mesh=_mesh(),
        out_type=jax.ShapeDtypeStruct((2, n, 16), jnp.float32),
        compiler_params=pltpu.CompilerParams(use_tc_tiling_on_sc=False),
        scratch_types=[
            pltpu.VMEM((nch, _CH), jnp.int32),
            pltpu.VMEM((_CH, 16), jnp.float32),
            pltpu.VMEM_SHARED((n, 16), jnp.float32),
        ] + [pltpu.SemaphoreType.DMA] * _K,
    )
    def deg_k(ei_hbm, zeros_hbm, ones_hbm, out_hbm, idx_v, ones_v,
              acc_sh, *sems):
        c = lax.axis_index("c")
        s = lax.axis_index("s")
        pltpu.sync_copy(ei_hbm.at[c, s], idx_v)
        pltpu.sync_copy(ones_hbm, ones_v)
        pltpu.sync_copy(zeros_hbm.at[s], acc_sh.at[pl.ds(s * rpt, rpt), :])
        plsc.subcore_barrier()

        def body(i, carry):
            descs = []
            for b in range(_K):
                descs.append(
                    pltpu.async_copy(
                        ones_v, acc_sh.at[idx_v.at[i * _K + b]], sems[b], add=True
                    )
                )
            for b in range(_K):
                descs[b].wait()
            return carry

        lax.fori_loop(0, nch // _K, body, 0)
        plsc.subcore_barrier()
        pltpu.sync_copy(acc_sh.at[pl.ds(s * rpt, rpt), :],
                        out_hbm.at[c, pl.ds(s * rpt, rpt), :])

    return deg_k


@functools.cache
def _make_agg(n, e, d):
    hd = d // _NC
    nch = e // (_NS * _CH)
    rpt = n // _NS
    niter = nch // _K
    assert e % (_NS * _CH) == 0 and d % _NC == 0 and nch % _K == 0

    nwb = rpt // _CH
    assert rpt % _CH == 0

    @functools.partial(
        pl.kernel,
        mesh=_mesh(),
        out_type=jax.ShapeDtypeStruct((_NC * n, hd), jnp.float32),
        compiler_params=pltpu.CompilerParams(use_tc_tiling_on_sc=False),
        scratch_types=[
            pltpu.VMEM((nch, _CH), jnp.int32),
            pltpu.VMEM((nch, _CH), jnp.int32),
            pltpu.VMEM((_K, _CH, hd), jnp.float32),
            pltpu.VMEM((nwb, _CH), jnp.int32),
            pltpu.VMEM_SHARED((n, hd), jnp.float32),
        ] + [pltpu.SemaphoreType.DMA] * (2 * _K),
    )
    def agg_k(h_hbm, si_hbm, ei_hbm, wi_hbm, zeros_hbm, out_hbm,
              sidx, didx, rows, widx, acc_sh, *sems):
        gsem = sems[:_K]
        ssem = sems[_K:]
        c = lax.axis_index("c")
        s = lax.axis_index("s")
        h_view = h_hbm
        pltpu.sync_copy(si_hbm.at[c, s], sidx)
        pltpu.sync_copy(ei_hbm.at[1, s], didx)
        pltpu.sync_copy(wi_hbm.at[c, s], widx)
        pltpu.sync_copy(zeros_hbm.at[s], acc_sh.at[pl.ds(s * rpt, rpt), :])
        plsc.subcore_barrier()

        for b in range(_K):
            pltpu.async_copy(h_view.at[sidx.at[b]], rows.at[b], gsem[b])

        def body(i, carry):
            for b in range(_K):
                k = i * _K + b
                pltpu.make_async_copy(
                    h_view.at[sidx.at[k]], rows.at[b], gsem[b]
                ).wait()
                pltpu.async_copy(
                    rows.at[b], acc_sh.at[didx.at[k]], ssem[b], add=True
                )
            for b in range(_K):
                k = i * _K + b
                pltpu.make_async_copy(
                    rows.at[b], acc_sh.at[didx.at[k]], ssem[b]
                ).wait()
                @pl.when(i + 1 < niter)
                def _():
                    pltpu.async_copy(
                        h_view.at[sidx.at[k + _K]], rows.at[b], gsem[b]
                    )
            return carry

        lax.fori_loop(0, niter, body, 0)
        plsc.subcore_barrier()
        for j in range(nwb):
            pltpu.sync_copy(acc_sh.at[pl.ds(s * rpt + j * _CH, _CH), :],
                            rows.at[j])
            pltpu.async_copy(rows.at[j], out_hbm.at[widx.at[j]], gsem[j])
        for j in range(nwb):
            pltpu.make_async_copy(
                rows.at[j], out_hbm.at[widx.at[j]], gsem[j]
            ).wait()

    return agg_k


def _prep_call(x, deg):
    n, d = x.shape

    def body(x_ref, d_ref, o_ref):
        nsrc = lax.rsqrt(jnp.maximum(d_ref[0][:, 0:1], 1.0))
        o_ref[:, :] = x_ref[:, :] * nsrc

    return pl.pallas_call(
        body, out_shape=jax.ShapeDtypeStruct((n, d), jnp.float32)
    )(x, deg)


def _dense_call(p, deg, w, b, g, be, al, split_out):
    n, d = p.shape

    def body(p_ref, d_ref, wr, br, gr, ber, alr, o_ref):
        nd = lax.rsqrt(jnp.maximum(d_ref[1][:, 0:1], 1.0))
        y = jnp.dot(p_ref[:, :] * nd, wr[:, :],
                    preferred_element_type=jnp.float32) + br[:, :]
        m = jnp.mean(y, axis=0, keepdims=True)
        yc = y - m
        v = jnp.mean(yc * yc, axis=0, keepdims=True)
        y = gr[:, :] * yc * lax.rsqrt(v + 1e-5) + ber[:, :]
        y = jnp.where(y >= 0.0, y, alr[0, 0] * y)
        if split_out:
            y = y * lax.rsqrt(jnp.maximum(d_ref[0][:, 0:1], 1.0))
        o_ref[:, :] = y

    return pl.pallas_call(
        body, out_shape=jax.ShapeDtypeStruct((n, d), jnp.float32)
    )(p, deg, w, b, g, be, al)


def kernel(x, edge_index, W1, b1, g1, be1, a1, W2, b2, g2, be2, a2):
    n, d = x.shape
    e = edge_index.shape[1]
    nch = e // (_NS * _CH)
    rpt = n // _NS
    hd = d // _NC

    ei4 = jnp.reshape(edge_index, (2, _NS, nch, _CH))
    s2 = ei4[0] * 2
    si4 = jnp.stack([s2, s2 + 1])
    v2 = 2 * jnp.arange(n, dtype=jnp.int32).reshape(_NS, rpt // _CH, _CH)
    wi4 = jnp.stack([v2, v2 + 1])
    zeros_h = jnp.zeros((_NS, rpt, hd), jnp.float32)
    zeros16 = jnp.zeros((_NS, rpt, 16), jnp.float32)
    ones16 = jnp.ones((_CH, 16), jnp.float32)

    deg = _make_deg(n, e)(ei4, zeros16, ones16)

    b1r, g1r, be1r = b1.reshape(1, d), g1.reshape(1, d), be1.reshape(1, d)
    b2r, g2r, be2r = b2.reshape(1, d), g2.reshape(1, d), be2.reshape(1, d)
    a1r, a2r = a1.reshape(1, 1), a2.reshape(1, 1)

    agg = _make_agg(n, e, d)
    h = _prep_call(x, deg).reshape(_NC * n, hd)
    p = agg(h, si4, ei4, wi4, zeros_h).reshape(n, d)
    h = _dense_call(p, deg, W1, b1r, g1r, be1r, a1r, True).reshape(_NC * n, hd)
    p = agg(h, si4, ei4, wi4, zeros_h).reshape(n, d)
    out = _dense_call(p, deg, W2, b2r, g2r, be2r, a2r, False)
    return out


# speedup vs baseline: 13.6518x; 1.0210x over previous
import functools

import jax
import jax.numpy as jnp
from jax import lax
from jax.experimental import pallas as pl
from jax.experimental.pallas import tpu as pltpu
from jax.experimental.pallas import tpu_sc as plsc

_NC = 2
_NS = 16
_CH = 125
_K = 5


def _mesh():
    return plsc.VectorSubcoreMesh(core_axis_name="c", subcore_axis_name="s")


@functools.cache
def _make_deg(n, e):
    nch = e // (_NS * _CH)
    rpt = n // _NS
    assert e % (_NS * _CH) == 0 and n % _NS == 0 and nch % _K == 0
    @functools.partial(
        pl.kernel,
        mesh=_mesh(),
        out_type=jax.ShapeDtypeStruct((2, n, 16), jnp.float32),
        compiler_params=pltpu.CompilerParams(use_tc_tiling_on_sc=False),
        scratch_types=[
            pltpu.VMEM((nch, _CH), jnp.int32),
            pltpu.VMEM((_CH, 16), jnp.float32),
            pltpu.VMEM_SHARED((n, 16), jnp.float32),
        ] + [pltpu.SemaphoreType.DMA] * _K,
    )
    def deg_k(ei_hbm, zeros_hbm, ones_hbm, out_hbm, idx_v, ones_v,
              acc_sh, *sems):
        c = lax.axis_index("c")
        s = lax.axis_index("s")
        pltpu.sync_copy(ei_hbm.at[c, s], idx_v)
        pltpu.sync_copy(ones_hbm, ones_v)
        pltpu.sync_copy(zeros_hbm.at[s], acc_sh.at[pl.ds(s * rpt, rpt), :])
        plsc.subcore_barrier()

        def body(i, carry):
            descs = []
            for b in range(_K):
                descs.append(
                    pltpu.async_copy(
                        ones_v, acc_sh.at[idx_v.at[i * _K + b]], sems[b], add=True
                    )
                )
            for b in range(_K):
                descs[b].wait()
            return carry

        lax.fori_loop(0, nch // _K, body, 0)
        plsc.subcore_barrier()
        pltpu.sync_copy(acc_sh.at[pl.ds(s * rpt, rpt), :],
                        out_hbm.at[c, pl.ds(s * rpt, rpt), :])

    return deg_k


@functools.cache
def _make_agg(n, e, d):
    hd = d // _NC
    nch = e // (_NS * _CH)
    rpt = n // _NS
    niter = nch // _K
    assert e % (_NS * _CH) == 0 and d % _NC == 0 and nch % _K == 0

    nwb = rpt // _CH
    assert rpt % _CH == 0

    @functools.partial(
        pl.kernel,
        mesh=_mesh(),
        out_type=jax.ShapeDtypeStruct((_NC * n, hd), jnp.float32),
        compiler_params=pltpu.CompilerParams(use_tc_tiling_on_sc=False),
        scratch_types=[
            pltpu.VMEM((nch, _CH), jnp.int32),
            pltpu.VMEM((nch, _CH), jnp.int32),
            pltpu.VMEM((_K, _CH, hd), jnp.float32),
            pltpu.VMEM((nwb, _CH), jnp.int32),
            pltpu.VMEM_SHARED((n, hd), jnp.float32),
        ] + [pltpu.SemaphoreType.DMA] * (2 * _K),
    )
    def agg_k(h_hbm, si_hbm, ei_hbm, wi_hbm, zeros_hbm, out_hbm,
              sidx, didx, rows, widx, acc_sh, *sems):
        gsem = sems[:_K]
        ssem = sems[_K:]
        c = lax.axis_index("c")
        s = lax.axis_index("s")
        h_view = h_hbm
        d0 = pltpu.async_copy(si_hbm.at[c, s], sidx, gsem[0])
        d1 = pltpu.async_copy(ei_hbm.at[1, s], didx, gsem[1])
        d2 = pltpu.async_copy(wi_hbm.at[c, s], widx, gsem[2])
        d3 = pltpu.async_copy(zeros_hbm.at[s],
                              acc_sh.at[pl.ds(s * rpt, rpt), :], gsem[3])
        d0.wait()
        d1.wait()
        d2.wait()
        d3.wait()
        plsc.subcore_barrier()

        for b in range(_K):
            pltpu.async_copy(h_view.at[sidx.at[b]], rows.at[b], gsem[b])

        def body(i, carry):
            for b in range(_K):
                k = i * _K + b
                pltpu.make_async_copy(
                    h_view.at[sidx.at[k]], rows.at[b], gsem[b]
                ).wait()
                pltpu.async_copy(
                    rows.at[b], acc_sh.at[didx.at[k]], ssem[b], add=True
                )
            for b in range(_K):
                k = i * _K + b
                pltpu.make_async_copy(
                    rows.at[b], acc_sh.at[didx.at[k]], ssem[b]
                ).wait()
                @pl.when(i + 1 < niter)
                def _():
                    pltpu.async_copy(
                        h_view.at[sidx.at[k + _K]], rows.at[b], gsem[b]
                    )
            return carry

        lax.fori_loop(0, niter, body, 0)
        plsc.subcore_barrier()
        for j in range(nwb):
            pltpu.sync_copy(acc_sh.at[pl.ds(s * rpt + j * _CH, _CH), :],
                            rows.at[j])
            pltpu.async_copy(rows.at[j], out_hbm.at[widx.at[j]], gsem[j])
        for j in range(nwb):
            pltpu.make_async_copy(
                rows.at[j], out_hbm.at[widx.at[j]], gsem[j]
            ).wait()

    return agg_k


def _prep_call(x, deg):
    n, d = x.shape

    def body(x_ref, d_ref, o_ref):
        nsrc = lax.rsqrt(jnp.maximum(d_ref[0][:, 0:1], 1.0))
        o_ref[:, :] = x_ref[:, :] * nsrc

    return pl.pallas_call(
        body, out_shape=jax.ShapeDtypeStruct((n, d), jnp.float32)
    )(x, deg)


def _dense_call(p, deg, w, b, g, be, al, split_out):
    n, d = p.shape

    def body(p_ref, d_ref, wr, br, gr, ber, alr, o_ref):
        nd = lax.rsqrt(jnp.maximum(d_ref[1][:, 0:1], 1.0))
        y = jnp.dot(p_ref[:, :] * nd, wr[:, :],
                    preferred_element_type=jnp.float32) + br[:, :]
        m = jnp.mean(y, axis=0, keepdims=True)
        yc = y - m
        v = jnp.mean(yc * yc, axis=0, keepdims=True)
        y = gr[:, :] * yc * lax.rsqrt(v + 1e-5) + ber[:, :]
        y = jnp.where(y >= 0.0, y, alr[0, 0] * y)
        if split_out:
            y = y * lax.rsqrt(jnp.maximum(d_ref[0][:, 0:1], 1.0))
        o_ref[:, :] = y

    return pl.pallas_call(
        body, out_shape=jax.ShapeDtypeStruct((n, d), jnp.float32)
    )(p, deg, w, b, g, be, al)


def kernel(x, edge_index, W1, b1, g1, be1, a1, W2, b2, g2, be2, a2):
    n, d = x.shape
    e = edge_index.shape[1]
    nch = e // (_NS * _CH)
    rpt = n // _NS
    hd = d // _NC

    ei4 = jnp.reshape(edge_index, (2, _NS, nch, _CH))
    s2 = ei4[0] * 2
    si4 = jnp.stack([s2, s2 + 1])
    v2 = 2 * jnp.arange(n, dtype=jnp.int32).reshape(_NS, rpt // _CH, _CH)
    wi4 = jnp.stack([v2, v2 + 1])
    zeros_h = jnp.zeros((_NS, rpt, hd), jnp.float32)
    zeros16 = jnp.zeros((_NS, rpt, 16), jnp.float32)
    ones16 = jnp.ones((_CH, 16), jnp.float32)

    deg = _make_deg(n, e)(ei4, zeros16, ones16)

    b1r, g1r, be1r = b1.reshape(1, d), g1.reshape(1, d), be1.reshape(1, d)
    b2r, g2r, be2r = b2.reshape(1, d), g2.reshape(1, d), be2.reshape(1, d)
    a1r, a2r = a1.reshape(1, 1), a2.reshape(1, 1)

    agg = _make_agg(n, e, d)
    h = _prep_call(x, deg).reshape(_NC * n, hd)
    p = agg(h, si4, ei4, wi4, zeros_h).reshape(n, d)
    h = _dense_call(p, deg, W1, b1r, g1r, be1r, a1r, True).reshape(_NC * n, hd)
    p = agg(h, si4, ei4, wi4, zeros_h).reshape(n, d)
    out = _dense_call(p, deg, W2, b2r, g2r, be2r, a2r, False)
    return out
